# baseline (speedup 1.0000x reference)
import jax
import jax.numpy as jnp
from jax.experimental import pallas as pl
from jax.experimental.pallas import tpu as pltpu


def _copy_body(ent_ref, rel_ref, ent_out, rel_out, ent_sem, rel_sem):
    ent_copy = pltpu.make_async_copy(ent_ref, ent_out, ent_sem)
    rel_copy = pltpu.make_async_copy(rel_ref, rel_out, rel_sem)
    ent_copy.start()
    rel_copy.start()
    ent_copy.wait()
    rel_copy.wait()


def kernel(x_dict, edge_index, entity_emb, rel_emb):
    entity_out, rel_out = pl.pallas_call(
        _copy_body,
        in_specs=[
            pl.BlockSpec(memory_space=pl.ANY),
            pl.BlockSpec(memory_space=pl.ANY),
        ],
        out_specs=[
            pl.BlockSpec(memory_space=pl.ANY),
            pl.BlockSpec(memory_space=pl.ANY),
        ],
        scratch_shapes=[pltpu.SemaphoreType.DMA, pltpu.SemaphoreType.DMA],
        out_shape=[
            jax.ShapeDtypeStruct(entity_emb.shape, entity_emb.dtype),
            jax.ShapeDtypeStruct(rel_emb.shape, rel_emb.dtype),
        ],
    )(entity_emb, rel_emb)
    return (entity_out, rel_out)


# speedup vs baseline: 12.7292x; 12.7292x over previous
import jax
import jax.numpy as jnp
from jax.experimental import pallas as pl
from jax.experimental.pallas import tpu as pltpu


def _copy_body(x_ref, o_ref):
    o_ref[...] = x_ref[...]


def _pallas_copy(x, block_rows):
    rows, cols = x.shape
    return pl.pallas_call(
        _copy_body,
        grid=(rows // block_rows,),
        in_specs=[pl.BlockSpec((block_rows, cols), lambda i: (i, 0))],
        out_specs=pl.BlockSpec((block_rows, cols), lambda i: (i, 0)),
        out_shape=jax.ShapeDtypeStruct(x.shape, x.dtype),
    )(x)


def kernel(x_dict, edge_index, entity_emb, rel_emb):
    entity_out = _pallas_copy(entity_emb, 2000)
    rel_out = _pallas_copy(rel_emb, 512)
    return (entity_out, rel_out)
